# baseline (speedup 1.0000x reference)
import functools

import jax
import jax.numpy as jnp
from jax import lax
from jax.experimental import pallas as pl
from jax.experimental.pallas import tpu as pltpu
from jax.experimental.pallas import tpu_sc as plsc

VOCAB = 1000000
EMBED = 32
BATCH = 16384
NEG = 20

NUM_CORES = 2
NUM_SUBCORES = 16
NW = NUM_CORES * NUM_SUBCORES
BPW = BATCH // NW
BC = 64
NBLK = BPW // BC
NEG_ROWS_BLK = BC * NEG
IDX_CHUNK = 128


def _sc_body(ww_h, wu_h, inp_h, tgt_h, neg_h, pos_h, negl_h,
             inp_v, tgt_v, neg_v, emb_v, ctx_v, nrow_v, pos_v, negl_v, sem):
    cid = lax.axis_index("c")
    sid = lax.axis_index("s")
    wid = sid * NUM_CORES + cid

    pltpu.sync_copy(inp_h.at[wid], inp_v)
    pltpu.sync_copy(tgt_h.at[wid], tgt_v)
    pltpu.sync_copy(neg_h.at[wid], neg_v)

    emb_2d = emb_v
    ctx_2d = ctx_v
    nrow_2d = nrow_v
    descs = []
    for j in range(BPW // IDX_CHUNK):
        dst = emb_2d.at[pl.ds(j * IDX_CHUNK, IDX_CHUNK)]
        descs.append(pltpu.async_copy(ww_h.at[inp_v.at[j]], dst, sem))
        dst = ctx_2d.at[pl.ds(j * IDX_CHUNK, IDX_CHUNK)]
        descs.append(pltpu.async_copy(wu_h.at[tgt_v.at[j]], dst, sem))
    for d in descs:
        d.wait()

    iota = lax.iota(jnp.int32, 16)

    @pl.loop(0, NBLK)
    def _block(blk):
        nchunks = NEG_ROWS_BLK // IDX_CHUNK
        nds = []
        for j in range(nchunks):
            dst = nrow_2d.at[pl.ds(j * IDX_CHUNK, IDX_CHUNK)]
            nds.append(pltpu.async_copy(wu_h.at[neg_v.at[blk * nchunks + j]],
                                        dst, sem))
        for d in nds:
            d.wait()

        @pl.loop(0, BC // 16)
        def _group(g):
            base = blk * BC + g * 16
            rows = base + iota
            nrow_base = (g * 16 + iota) * NEG
            acc_p = jnp.zeros((16,), jnp.float32)
            acc_n = jnp.zeros((16,), jnp.float32)
            for d in range(EMBED):
                cols = jnp.full((16,), d, jnp.int32)
                e = plsc.load_gather(emb_v, [rows, cols])
                c = plsc.load_gather(ctx_v, [rows, cols])
                acc_p = acc_p + e * c
                s = plsc.load_gather(nrow_v, [nrow_base, cols])
                for k in range(1, NEG):
                    s = s + plsc.load_gather(nrow_v, [nrow_base + k, cols])
                acc_n = acc_n + e * s
            pos_v[pl.ds(base, 16)] = acc_p
            negl_v[pl.ds(base, 16)] = -acc_n

    pltpu.sync_copy(pos_v, pos_h.at[wid])
    pltpu.sync_copy(negl_v, negl_h.at[wid])


def _make_sc_kernel():
    mesh = plsc.VectorSubcoreMesh(core_axis_name="c", subcore_axis_name="s")
    return pl.kernel(
        _sc_body,
        out_type=(
            jax.ShapeDtypeStruct((NW, BPW), jnp.float32),
            jax.ShapeDtypeStruct((NW, BPW), jnp.float32),
        ),
        mesh=mesh,
        scratch_types=(
            pltpu.VMEM((BPW // IDX_CHUNK, IDX_CHUNK), jnp.int32),
            pltpu.VMEM((BPW // IDX_CHUNK, IDX_CHUNK), jnp.int32),
            pltpu.VMEM((BPW * NEG // IDX_CHUNK, IDX_CHUNK), jnp.int32),
            pltpu.VMEM((BPW, EMBED), jnp.float32),
            pltpu.VMEM((BPW, EMBED), jnp.float32),
            pltpu.VMEM((NEG_ROWS_BLK, EMBED), jnp.float32),
            pltpu.VMEM((BPW,), jnp.float32),
            pltpu.VMEM((BPW,), jnp.float32),
            pltpu.SemaphoreType.DMA,
        ),
        compiler_params=pltpu.CompilerParams(
            needs_layout_passes=False, use_tc_tiling_on_sc=False),
    )


def _loss_body(pos_ref, negl_ref, out_ref):
    def logsig(x):
        return jnp.minimum(x, 0.0) - jnp.log1p(jnp.exp(-jnp.abs(x)))

    total = jnp.sum(logsig(pos_ref[...])) + jnp.sum(logsig(negl_ref[...]))
    out_ref[0, 0] = -total / BATCH


@jax.jit
def kernel(inputs, targets, neg_samples, W_w, W_u):
    inp_r = inputs.astype(jnp.int32).reshape(NW, BPW // IDX_CHUNK, IDX_CHUNK)
    tgt_r = targets.astype(jnp.int32).reshape(NW, BPW // IDX_CHUNK, IDX_CHUNK)
    neg_r = neg_samples.astype(jnp.int32).reshape(
        NW, BPW * NEG // IDX_CHUNK, IDX_CHUNK)

    pos, negl = _make_sc_kernel()(W_w, W_u, inp_r, tgt_r, neg_r)

    loss = pl.pallas_call(
        _loss_body,
        out_shape=jax.ShapeDtypeStruct((1, 1), jnp.float32),
        out_specs=pl.BlockSpec(memory_space=pltpu.SMEM),
    )(pos.reshape(128, 128), negl.reshape(128, 128))
    return loss[0, 0]
